# initial kernel scaffold (unmeasured)
import jax
import jax.numpy as jnp
from jax import lax
from jax.experimental import pallas as pl
from jax.experimental.pallas import tpu as pltpu

N_DEV = 4


def kernel(x, w_mat):
    m_per, k = x.shape
    n = w_mat.shape[1]
    n_per = n // N_DEV

    def body(x_ref, w_ref, out_ref, send_buf, recv_buf, send_sems, recv_sems):
        my = lax.axis_index("i")

        x_bf = x_ref[:, :].astype(jnp.bfloat16)
        w_bf = w_ref[:, :].astype(jnp.bfloat16)
        y = jnp.dot(x_bf, w_bf, preferred_element_type=jnp.float32)
        y_bf = y.astype(jnp.bfloat16)

        rdmas = []
        for d in range(1, N_DEV):
            p = (my + d) % N_DEV
            send_buf[d] = lax.dynamic_slice(y_bf, (0, p * n_per), (m_per, n_per))
            rdma = pltpu.make_async_remote_copy(
                src_ref=send_buf.at[d],
                dst_ref=recv_buf.at[d],
                send_sem=send_sems.at[d],
                recv_sem=recv_sems.at[d],
                device_id=(p,),
                device_id_type=pl.DeviceIdType.MESH,
            )
            rdma.start()
            rdmas.append(rdma)

        out_ref[pl.ds(my * m_per, m_per), :] = lax.dynamic_slice(
            y, (0, my * n_per), (m_per, n_per)
        )

        for d in range(1, N_DEV):
            rdmas[d - 1].wait_recv()
            s = (my - d) % N_DEV
            out_ref[pl.ds(s * m_per, m_per), :] = recv_buf[d].astype(jnp.float32)

        for rdma in rdmas:
            rdma.wait_send()

    out_shape = jax.ShapeDtypeStruct((N_DEV * m_per, n_per), jnp.float32)
    return pl.pallas_call(
        body,
        out_shape=out_shape,
        in_specs=[
            pl.BlockSpec(memory_space=pltpu.VMEM),
            pl.BlockSpec(memory_space=pltpu.VMEM),
        ],
        out_specs=pl.BlockSpec(memory_space=pltpu.VMEM),
        scratch_shapes=[
            pltpu.VMEM((N_DEV, m_per, n_per), jnp.bfloat16),
            pltpu.VMEM((N_DEV, m_per, n_per), jnp.bfloat16),
            pltpu.SemaphoreType.DMA((N_DEV,)),
            pltpu.SemaphoreType.DMA((N_DEV,)),
        ],
        compiler_params=pltpu.CompilerParams(collective_id=0),
    )(x, w_mat)


# baseline (device time: 15481 ns/iter reference)
import jax
import jax.numpy as jnp
from jax import lax
from jax.experimental import pallas as pl
from jax.experimental.pallas import tpu as pltpu

N_DEV = 4


def kernel(x, w_mat):
    m_per, k = x.shape
    n = w_mat.shape[1]
    n_per = n // N_DEV

    def body(x_ref, w_ref, out_ref, send_buf, recv_buf, send_sems, recv_sems):
        my = lax.axis_index("i")

        x_bf = x_ref[:, :].astype(jnp.bfloat16)
        w_bf = w_ref[:, :].astype(jnp.bfloat16)
        y = jnp.dot(x_bf, w_bf, preferred_element_type=jnp.float32)
        y_bf = y.astype(jnp.bfloat16)

        for j in range(N_DEV):
            send_buf[j] = y_bf[:, j * n_per:(j + 1) * n_per]

        rdmas = []
        for d in range(1, N_DEV):
            p = (my + d) % N_DEV
            rdma = pltpu.make_async_remote_copy(
                src_ref=send_buf.at[p],
                dst_ref=recv_buf.at[d],
                send_sem=send_sems.at[d],
                recv_sem=recv_sems.at[d],
                device_id=(p,),
                device_id_type=pl.DeviceIdType.MESH,
            )
            rdma.start()
            rdmas.append(rdma)

        out_ref[pl.ds(my * m_per, m_per), :] = send_buf[my].astype(jnp.float32)

        for d in range(1, N_DEV):
            rdmas[d - 1].wait_recv()
            s = (my - d) % N_DEV
            out_ref[pl.ds(s * m_per, m_per), :] = recv_buf[d].astype(jnp.float32)

        for rdma in rdmas:
            rdma.wait_send()

    out_shape = jax.ShapeDtypeStruct((N_DEV * m_per, n_per), jnp.float32)
    return pl.pallas_call(
        body,
        out_shape=out_shape,
        in_specs=[
            pl.BlockSpec(memory_space=pltpu.VMEM),
            pl.BlockSpec(memory_space=pltpu.VMEM),
        ],
        out_specs=pl.BlockSpec(memory_space=pltpu.VMEM),
        scratch_shapes=[
            pltpu.VMEM((N_DEV, m_per, n_per), jnp.bfloat16),
            pltpu.VMEM((N_DEV, m_per, n_per), jnp.bfloat16),
            pltpu.SemaphoreType.DMA((N_DEV,)),
            pltpu.SemaphoreType.DMA((N_DEV,)),
        ],
    )(x, w_mat)


# device time: 12850 ns/iter; 1.2047x vs baseline; 1.2047x over previous
import functools

import jax
import jax.numpy as jnp
from jax import lax
from jax.experimental import pallas as pl
from jax.experimental.pallas import tpu as pltpu

N_DEV = 4


def kernel(x, w_mat):
    m_per, k = x.shape
    n = w_mat.shape[1]
    n_per = n // N_DEV

    def body(x_ref, w_ref, out_ref, send_buf, recv_buf, send_sems, recv_sems):
        my = lax.axis_index("i")

        barrier_sem = pltpu.get_barrier_semaphore()
        for d in range(1, N_DEV):
            pl.semaphore_signal(
                barrier_sem, inc=1,
                device_id=((my + d) % N_DEV,),
                device_id_type=pl.DeviceIdType.MESH,
            )
        pl.semaphore_wait(barrier_sem, N_DEV - 1)

        x_bf = x_ref[:, :].astype(jnp.bfloat16)

        def send_desc(j, d):
            return pltpu.make_async_remote_copy(
                src_ref=send_buf.at[j],
                dst_ref=recv_buf.at[d],
                send_sem=send_sems.at[j],
                recv_sem=recv_sems.at[d],
                device_id=(j,),
                device_id_type=pl.DeviceIdType.MESH,
            )

        for j in range(N_DEV):
            w_bf = w_ref[:, j * n_per:(j + 1) * n_per].astype(jnp.bfloat16)
            y_j = jnp.dot(x_bf, w_bf, preferred_element_type=jnp.float32)
            send_buf[j] = y_j.astype(jnp.bfloat16)
            d = (j - my) % N_DEV

            @pl.when(j != my)
            def _():
                send_desc(j, d).start()

        out_ref[pl.ds(my * m_per, m_per), :] = send_buf[my].astype(jnp.float32)

        for d in range(1, N_DEV):
            recv = pltpu.make_async_remote_copy(
                src_ref=send_buf.at[0],
                dst_ref=recv_buf.at[d],
                send_sem=send_sems.at[0],
                recv_sem=recv_sems.at[d],
                device_id=(0,),
                device_id_type=pl.DeviceIdType.MESH,
            )
            recv.wait_recv()
            s = (my - d) % N_DEV
            out_ref[pl.ds(s * m_per, m_per), :] = recv_buf[d].astype(jnp.float32)

        for j in range(N_DEV):
            @pl.when(j != my)
            def _():
                send_desc(j, (j - my) % N_DEV).wait_send()

    out_shape = jax.ShapeDtypeStruct((N_DEV * m_per, n_per), jnp.float32)
    return pl.pallas_call(
        body,
        out_shape=out_shape,
        in_specs=[
            pl.BlockSpec(memory_space=pltpu.VMEM),
            pl.BlockSpec(memory_space=pltpu.VMEM),
        ],
        out_specs=pl.BlockSpec(memory_space=pltpu.VMEM),
        scratch_shapes=[
            pltpu.VMEM((N_DEV, m_per, n_per), jnp.bfloat16),
            pltpu.VMEM((N_DEV, m_per, n_per), jnp.bfloat16),
            pltpu.SemaphoreType.DMA((N_DEV,)),
            pltpu.SemaphoreType.DMA((N_DEV,)),
        ],
        compiler_params=pltpu.CompilerParams(collective_id=0),
    )(x, w_mat)
